# baseline (speedup 1.0000x reference)
import jax
import jax.numpy as jnp
from jax.experimental import pallas as pl

_THR = 0.2
_FILL = -1000000000.0
_GONE = -2000000000.0
_K = 20
_RPC = 8


def _nms_topk_kernel(x_ref, out_ref):
    cpb, h, w = x_ref.shape
    nch = h // _RPC
    neg = jnp.float32(-jnp.inf)

    cms = []
    for p in range(cpb):
        x = x_ref[p]
        colpad = jnp.full((h, 1), neg, jnp.float32)
        left = jnp.concatenate([colpad, x[:, :-1]], axis=1)
        right = jnp.concatenate([x[:, 1:], colpad], axis=1)
        hmax = jnp.maximum(left, right)
        h3 = jnp.maximum(hmax, x)
        rowpad = jnp.full((1, w), neg, jnp.float32)
        above = jnp.concatenate([rowpad, h3[:-1, :]], axis=0)
        below = jnp.concatenate([h3[1:, :], rowpad], axis=0)
        nmax = jnp.maximum(jnp.maximum(hmax, jnp.float32(_THR)),
                           jnp.maximum(above, below))
        masked = jnp.where(x > nmax, x, jnp.float32(_FILL))
        x_ref[p] = masked
        cm = jnp.max(masked.reshape(nch, _RPC, w), axis=(1, 2))
        cms.append(cm.reshape(1, nch))

    cm8 = jnp.concatenate(cms, axis=0)
    chiota = jax.lax.broadcasted_iota(jnp.int32, (cpb, nch), 1)
    liota3 = (jax.lax.broadcasted_iota(jnp.int32, (1, _RPC, w), 1) * 512
              + jax.lax.broadcasted_iota(jnp.int32, (1, _RPC, w), 2))
    oiota = jax.lax.broadcasted_iota(jnp.int32, (cpb, 128), 1)
    big = jnp.int32(1 << 30)
    vvec = jnp.zeros((cpb, 128), jnp.float32)
    xvec = jnp.zeros((cpb, 128), jnp.float32)
    yvec = jnp.zeros((cpb, 128), jnp.float32)
    for i in range(_K):
        m = jnp.max(cm8, axis=1, keepdims=True)
        cmask = cm8 == m
        ch_v = jnp.min(jnp.where(cmask, chiota, big), axis=1, keepdims=True)
        chunk8 = jnp.concatenate(
            [x_ref[p, pl.ds(ch_v[p, 0] * _RPC, _RPC), :][None]
             for p in range(cpb)], axis=0)
        sel = chiota == ch_v
        m3 = m[:, :, None]
        fl3 = jnp.min(jnp.where(chunk8 == m3, liota3, big),
                      axis=(1, 2), keepdims=True)
        fl = fl3[:, :, 0]
        col = jnp.bitwise_and(fl, 511)
        row = ch_v * _RPC + (fl >> 9)
        vvec = jnp.where(oiota == i, m, vvec)
        xvec = jnp.where(oiota == i, col.astype(jnp.float32), xvec)
        yvec = jnp.where(oiota == i, row.astype(jnp.float32), yvec)
        new8 = jnp.where(liota3 == fl3, jnp.float32(_GONE), chunk8)
        for p in range(cpb):
            x_ref[p, pl.ds(ch_v[p, 0] * _RPC, _RPC), :] = new8[p]
        newmax = jnp.max(new8, axis=(1, 2), keepdims=True)[:, :, 0]
        cm8 = jnp.where(sel, newmax, cm8)
    out_ref[:, 0, :] = vvec
    out_ref[:, 1, :] = xvec
    out_ref[:, 2, :] = yvec


def kernel(confmaps, k):
    b, n, h, w = confmaps.shape
    planes = b * n
    cpb = 32 if planes % 32 == 0 else 1
    x = confmaps.reshape(planes, h, w)
    out = pl.pallas_call(
        _nms_topk_kernel,
        grid=(planes // cpb,),
        in_specs=[pl.BlockSpec((cpb, h, w), lambda i: (i, 0, 0))],
        out_specs=pl.BlockSpec((cpb, 8, 128), lambda i: (i, 0, 0)),
        out_shape=jax.ShapeDtypeStruct((planes, 8, 128), jnp.float32),
    )(x)
    vals = out[:, 0, :_K].reshape(b, n, _K)
    xcoord = out[:, 1, :_K].reshape(b, n, _K)
    ycoord = out[:, 2, :_K].reshape(b, n, _K)
    peaks = jnp.stack([xcoord, ycoord], axis=-1)
    valid = vals > jnp.float32(_THR)
    return peaks, vals, valid
